# baseline (speedup 1.0000x reference)
import functools

import jax
import jax.numpy as jnp
from jax import lax
from jax.experimental import pallas as pl
from jax.experimental.pallas import tpu as pltpu
from jax.experimental.pallas import tpu_sc as plsc

NC, NS = 2, 16
NW = NC * NS
D = 128
_MAX_CARD, _MIN_CARD = 20.0, 0.0
_EXACT = lax.Precision.HIGHEST
_LIN = lax.Precision.DEFAULT


def _dot(a, b, prec):
    return jnp.dot(a, b, preferred_element_type=jnp.float32, precision=prec)


def _dott(a, b, prec):
    return lax.dot_general(a, b, (((0,), (0,)), ((), ())),
                           preferred_element_type=jnp.float32, precision=prec)


@functools.lru_cache(maxsize=None)
def _make_edge_agg(n_nodes, n_edges, chunk):
    assert n_edges % (NW * chunk) == 0 and n_nodes % NS == 0
    nchunk = n_edges // (NW * chunk)
    assert nchunk % 2 == 0
    rpt = n_nodes // NS
    assert rpt % chunk == 0
    nslab = rpt // chunk
    mesh = plsc.VectorSubcoreMesh(core_axis_name="c", subcore_axis_name="s")

    @functools.partial(
        pl.kernel,
        out_type=jax.ShapeDtypeStruct((NC, n_nodes, D), jnp.float32),
        mesh=mesh,
        scratch_types=[
            pltpu.VMEM((nchunk, chunk), jnp.int32),
            pltpu.VMEM((nchunk, chunk), jnp.int32),
            pltpu.VMEM((chunk, D), jnp.float32),
            pltpu.VMEM((chunk, D), jnp.float32),
            pltpu.VMEM_SHARED((n_nodes, D), jnp.float32),
            pltpu.SemaphoreType.DMA,
            pltpu.SemaphoreType.DMA,
            pltpu.SemaphoreType.DMA,
            pltpu.SemaphoreType.DMA,
        ],
    )
    def edge_agg(h_hbm, src_hbm, dst_hbm, zeros_hbm, out_hbm,
                 src_v, dst_v, rows0, rows1, agg_sh, sem0, sem1, sem2, sem3):
        cid = lax.axis_index("c")
        sid = lax.axis_index("s")
        wid = sid * NC + cid

        def slab(t):
            return pl.ds(sid * rpt + t * chunk, chunk)

        pltpu.sync_copy(zeros_hbm, rows0)
        for t in range(nslab):
            pltpu.async_copy(rows0, agg_sh.at[slab(t)], sem1)
        for t in range(nslab):
            pltpu.make_async_copy(rows0, agg_sh.at[slab(t)], sem1).wait()
        pltpu.sync_copy(src_hbm.at[wid], src_v)
        pltpu.sync_copy(dst_hbm.at[wid], dst_v)
        plsc.subcore_barrier()

        pltpu.async_copy(h_hbm.at[src_v.at[0]], rows0, sem0)
        nh = nchunk // 2

        def body(k, _):
            j0 = 2 * k
            pltpu.make_async_copy(h_hbm.at[src_v.at[j0]], rows0, sem0).wait()
            pltpu.async_copy(h_hbm.at[src_v.at[j0 + 1]], rows1, sem1)
            pltpu.async_copy(rows0, agg_sh.at[dst_v.at[j0]], sem2, add=True)
            pltpu.make_async_copy(
                h_hbm.at[src_v.at[j0 + 1]], rows1, sem1).wait()
            pltpu.async_copy(rows1, agg_sh.at[dst_v.at[j0 + 1]], sem3,
                             add=True)
            pltpu.make_async_copy(
                rows0, agg_sh.at[dst_v.at[j0]], sem2).wait()

            @pl.when(k + 1 < nh)
            def _issue_next():
                pltpu.async_copy(h_hbm.at[src_v.at[j0 + 2]], rows0, sem0)

            pltpu.make_async_copy(
                rows1, agg_sh.at[dst_v.at[j0 + 1]], sem3).wait()
            return _

        lax.fori_loop(0, nh, body, None)
        plsc.subcore_barrier()

        for t in range(nslab):
            b, sb = (rows0, sem0) if t % 2 == 0 else (rows1, sem1)
            if t >= 2:
                pltpu.make_async_copy(
                    b, out_hbm.at[cid, slab(t - 2)], sb).wait()
            pltpu.sync_copy(agg_sh.at[slab(t)], b)
            pltpu.async_copy(b, out_hbm.at[cid, slab(t)], sb)
        for t in range(max(nslab - 2, 0), nslab):
            b, sb = (rows0, sem0) if t % 2 == 0 else (rows1, sem1)
            pltpu.make_async_copy(b, out_hbm.at[cid, slab(t)], sb).wait()

    return edge_agg


@functools.lru_cache(maxsize=None)
def _make_row_gather(n_tab, n_idx):
    assert n_idx % (8 * NW) == 0
    per_w = n_idx // NW
    mesh = plsc.VectorSubcoreMesh(core_axis_name="c", subcore_axis_name="s")

    @functools.partial(
        pl.kernel,
        out_type=jax.ShapeDtypeStruct((n_idx, D), jnp.float32),
        mesh=mesh,
        scratch_types=[
            pltpu.VMEM((per_w,), jnp.int32),
            pltpu.VMEM((per_w, D), jnp.float32),
            pltpu.SemaphoreType.DMA,
        ],
    )
    def row_gather(tab_hbm, idx_hbm, out_hbm, idx_v, rows_v, sem):
        wid = lax.axis_index("s") * NC + lax.axis_index("c")
        base = wid * per_w
        pltpu.sync_copy(idx_hbm.at[pl.ds(base, per_w)], idx_v)
        pltpu.async_copy(tab_hbm.at[idx_v], rows_v, sem).wait()
        pltpu.sync_copy(rows_v, out_hbm.at[pl.ds(base, per_w)])

    return row_gather


def _mlp_sum(h, p, w1, b1, w2, b2, block):
    n = h.shape[0]

    def body(h_ref, p_ref, w1_ref, b1_ref, w2_ref, b2_ref, o_ref):
        u = h_ref[...] + p_ref[0] + p_ref[1]
        t = jnp.maximum(_dot(u, w1_ref[...], _LIN) + b1_ref[...], 0.0)
        o_ref[...] = _dot(t, w2_ref[...], _LIN) + b2_ref[...]

    return pl.pallas_call(
        body,
        grid=(n // block,),
        in_specs=[
            pl.BlockSpec((block, D), lambda i: (i, 0)),
            pl.BlockSpec((2, block, D), lambda i: (0, i, 0)),
            pl.BlockSpec((D, D), lambda i: (0, 0)),
            pl.BlockSpec((1, D), lambda i: (0, 0)),
            pl.BlockSpec((D, D), lambda i: (0, 0)),
            pl.BlockSpec((1, D), lambda i: (0, 0)),
        ],
        out_specs=pl.BlockSpec((block, D), lambda i: (i, 0)),
        out_shape=jax.ShapeDtypeStruct((n, D), jnp.float32),
    )(h, p, w1, b1.reshape(1, D), w2, b2.reshape(1, D))


def _interactor(itg, grows, dst8, w1, b1, w2, b2, l2w, l2b, nq, ng, block):
    n = itg.shape[0]
    ne = grows.shape[0]

    def body(it_ref, g_ref, d_ref, w1_ref, b1_ref, w2_ref, b2_ref,
             l2w_ref, l2b_ref, o_ref, m_ref):
        i = pl.program_id(0)
        dst = d_ref[...][:, 0:1]
        rows = i * block + lax.broadcasted_iota(jnp.int32, (ne, block), 1)
        oh = (dst == rows).astype(jnp.float32)
        agg = _dott(oh, g_ref[...], _EXACT)
        u = it_ref[...] + agg
        t = jnp.maximum(_dot(u, w1_ref[...], _LIN) + b1_ref[...], 0.0)
        y = _dot(t, w2_ref[...], _LIN) + b2_ref[...]
        o_ref[...] = y
        z = _dot(y, l2w_ref[...], _LIN) + l2b_ref[...]
        rid = i * block + lax.broadcasted_iota(jnp.int32, (block, 1), 0)
        msk = (rid >= nq).astype(jnp.float32)
        part = jnp.sum(z * msk, axis=0, keepdims=True) * (1.0 / ng)

        @pl.when(i == 0)
        def _():
            m_ref[...] = part

        @pl.when(i > 0)
        def _():
            m_ref[...] += part

    return pl.pallas_call(
        body,
        grid=(n // block,),
        in_specs=[
            pl.BlockSpec((block, D), lambda i: (i, 0)),
            pl.BlockSpec((ne, D), lambda i: (0, 0)),
            pl.BlockSpec((ne, 8), lambda i: (0, 0)),
            pl.BlockSpec((D, D), lambda i: (0, 0)),
            pl.BlockSpec((1, D), lambda i: (0, 0)),
            pl.BlockSpec((D, D), lambda i: (0, 0)),
            pl.BlockSpec((1, D), lambda i: (0, 0)),
            pl.BlockSpec((D, D), lambda i: (0, 0)),
            pl.BlockSpec((1, D), lambda i: (0, 0)),
        ],
        out_specs=[
            pl.BlockSpec((block, D), lambda i: (i, 0)),
            pl.BlockSpec((1, D), lambda i: (0, 0)),
        ],
        out_shape=[
            jax.ShapeDtypeStruct((n, D), jnp.float32),
            jax.ShapeDtypeStruct((1, D), jnp.float32),
        ],
    )(itg, grows, dst8, w1, b1.reshape(1, D), w2, b2.reshape(1, D),
      l2w, l2b.reshape(1, D))


def _query_gin1(xq, eq0b, eq1b, w1, b1, w2, b2):
    ne = eq0b.shape[0]
    nq = xq.shape[0]

    def body(x_ref, e0_ref, e1_ref, w1_ref, b1_ref, w2_ref, b2_ref, o_ref):
        src = e0_ref[...][:, 0:1]
        dst = e1_ref[...][:, 0:1]
        cols = lax.broadcasted_iota(jnp.int32, (ne, nq), 1)
        oh_s = (src == cols).astype(jnp.float32)
        oh_d = (dst == cols).astype(jnp.float32)
        x = x_ref[...]
        g = _dot(oh_s, x, _EXACT)
        agg = _dott(oh_d, g, _EXACT)
        u = x + agg
        t = jnp.maximum(_dot(u, w1_ref[...], _LIN) + b1_ref[...], 0.0)
        o_ref[...] = _dot(t, w2_ref[...], _LIN) + b2_ref[...]

    return pl.pallas_call(
        body,
        out_shape=jax.ShapeDtypeStruct((nq, D), jnp.float32),
    )(xq, eq0b, eq1b, w1, b1.reshape(1, D), w2, b2.reshape(1, D))


def _query_gin2(hq, mqb, eq0b, eq1b, w1, b1, w2, b2):
    ne = eq0b.shape[0]
    nq = hq.shape[0]
    nm = mqb.shape[1]

    def body(h_ref, mq_ref, e0_ref, e1_ref, w1_ref, b1_ref, w2_ref, b2_ref,
             o_ref):
        rid = lax.broadcasted_iota(jnp.int32, (nq, nm), 0)
        hits = (mq_ref[...] == rid).astype(jnp.float32)
        cnt = jnp.sum(hits, axis=1, keepdims=True)
        msk = (cnt == 0.0).astype(jnp.float32)
        hm = h_ref[...] * msk
        src = e0_ref[...][:, 0:1]
        dst = e1_ref[...][:, 0:1]
        cols = lax.broadcasted_iota(jnp.int32, (ne, nq), 1)
        oh_s = (src == cols).astype(jnp.float32)
        oh_d = (dst == cols).astype(jnp.float32)
        g = _dot(oh_s, hm, _EXACT)
        agg = _dott(oh_d, g, _EXACT)
        u = hm + agg
        t = jnp.maximum(_dot(u, w1_ref[...], _LIN) + b1_ref[...], 0.0)
        y = _dot(t, w2_ref[...], _LIN) + b2_ref[...]
        z = hm + y
        o_ref[...] = z * jax.nn.sigmoid(z)

    return pl.pallas_call(
        body,
        out_shape=jax.ShapeDtypeStruct((nq, D), jnp.float32),
    )(hq, mqb, eq0b, eq1b, w1, b1.reshape(1, D), w2, b2.reshape(1, D))


def _tail(hq3, readout_g, subq, ovn, sk0b, sk1b, tp):
    nsub, sublen = subq.shape
    nov, ovlen = ovn.shape
    nsk_e = sk0b.shape[0]

    def body(hq_ref, rg_ref, sq_ref, ov_ref, s0_ref, s1_ref,
             l1w, l1b, l3w, l3b,
             a1wa, a1wb, a1b, a2gw, a2gb,
             wwa, wwb, wwbias, g2wa, g2wb, g2b, w2wa, w2wl, w2b,
             pj_ref, o1_ref):
        hq = hq_ref[...]
        readout_g = rg_ref[...]
        z1 = _dot(hq, l1w[...], _LIN) + l1b[...]
        aq = jnp.zeros((nsub, D), jnp.float32)
        colsq = lax.broadcasted_iota(jnp.int32, (nsub, D), 1)
        for j in range(sublen):
            aq = aq + (sq_ref[...][:, j:j + 1] == colsq).astype(jnp.float32)
        readout_q = _dot(aq * (1.0 / sublen), z1, _EXACT)
        z3 = _dot(hq, l3w[...], _LIN) + l3b[...]
        ao = jnp.zeros((nov, D), jnp.float32)
        colso = lax.broadcasted_iota(jnp.int32, (nov, D), 1)
        for j in range(ovlen):
            ao = ao + (ov_ref[...][:, j:j + 1] == colso).astype(jnp.float32)
        ov_feat = _dot(ao * (1.0 / ovlen), z3, _EXACT)
        colss = lax.broadcasted_iota(jnp.int32, (nsk_e, nsub), 1)
        g0 = (s0_ref[...][:, 0:1] == colss).astype(jnp.float32)
        g1 = (s1_ref[...][:, 0:1] == colss).astype(jnp.float32)
        gsum = g0 + g1
        ovf = _dott(gsum, ov_feat, _EXACT)
        cnts = 1.0 + _dott(gsum, jnp.ones((nsk_e, 1), jnp.float32), _EXACT)
        ovf = ovf / cnts
        xg_l = _dot(g0, readout_q, _EXACT)
        xg_r = _dot(g0, ovf, _EXACT)
        agg_l = _dott(g1, xg_l, _EXACT)
        agg_r = _dott(g1, xg_r, _EXACT)
        u_l = readout_q + agg_l
        u_r = ovf + agg_r
        t = (_dot(u_l, a1wa[...], _LIN) + _dot(u_r, a1wb[...], _LIN)
             + a1b[...])
        t = jnp.maximum(t, 0.0)
        ovl2 = _dot(t, a2gw[...], _LIN) + a2gb[...]
        wl = (_dot(readout_q, wwa[...], _LIN) + _dot(ovl2, wwb[...], _LIN)
              + wwbias[...])
        wmax = jnp.max(wl, axis=0, keepdims=True)
        we = jnp.exp(wl - wmax)
        wsm = we / jnp.sum(we, axis=0, keepdims=True)
        hsk = readout_q * wsm
        rq2 = jnp.mean(hsk, axis=0, keepdims=True)
        nsk = jnp.sqrt(jnp.sum(hsk * hsk, axis=1, keepdims=True))
        s = jnp.sum(hsk * readout_g, axis=1, keepdims=True)
        anyz = jnp.max((nsk == 0.0).astype(jnp.float32), axis=0,
                       keepdims=True)
        pj = jnp.where(anyz > 0.5, jnp.zeros_like(s),
                       s / jnp.where(nsk == 0.0, 1.0, nsk))
        pj_ref[...] = pj
        swl = rq2 * jax.nn.sigmoid(rq2)
        swr = readout_g * jax.nn.sigmoid(readout_g)
        ro = (_dot(swl, g2wa[...], _LIN) + _dot(swr, g2wb[...], _LIN)
              + g2b[...])
        o = _dot(ro, w2wa[...], _LIN) + 8192.0 * w2wl[...] + w2b[...]
        o1_ref[...] = _MIN_CARD + (_MAX_CARD - _MIN_CARD) * jax.nn.sigmoid(o)

    return pl.pallas_call(
        body,
        out_shape=[
            jax.ShapeDtypeStruct((nsub, 1), jnp.float32),
            jax.ShapeDtypeStruct((1, 1), jnp.float32),
        ],
    )(hq3, readout_g, subq, ovn, sk0b, sk1b,
      tp["l1w"], tp["l1b"], tp["l3w"], tp["l3b"],
      tp["a1wa"], tp["a1wb"], tp["a1b"], tp["a2gw"], tp["a2gb"],
      tp["wwa"], tp["wwb"], tp["wwbias"],
      tp["g2wa"], tp["g2wb"], tp["g2b"], tp["w2wa"], tp["w2wl"], tp["w2b"])


def kernel(xg, eg, xq, eq, itedge, npairs, match_q, subqueries,
           skeleton_edges, overlap_nodes, params):
    ng, nq = xg.shape[0], xq.shape[0]
    n_eg = eg.shape[1]
    n_it = itedge.shape[1]
    n_np = npairs.shape[0]
    nit_nodes = nq + ng

    chunk = 128
    nchunk = n_eg // (NW * chunk)
    src_r = eg[0].reshape(NW, nchunk, chunk)
    dst_r = eg[1].reshape(NW, nchunk, chunk)
    zeros_g = jnp.zeros((chunk, D), jnp.float32)

    gp = params["graph_gnn"]
    edge_agg = _make_edge_agg(ng, n_eg, chunk)

    p1 = edge_agg(xg, src_r, dst_r, zeros_g)
    h1 = _mlp_sum(xg, p1, gp[0]["lin1"]["W"], gp[0]["lin1"]["b"],
                  gp[0]["lin2"]["W"], gp[0]["lin2"]["b"], block=512)
    p2 = edge_agg(h1, src_r, dst_r, zeros_g)
    hg = _mlp_sum(h1, p2, gp[1]["lin1"]["W"], gp[1]["lin1"]["b"],
                  gp[1]["lin2"]["W"], gp[1]["lin2"]["b"], block=512)

    eq0b = jnp.broadcast_to(eq[0][:, None], (eq.shape[1], 8)).astype(jnp.int32)
    eq1b = jnp.broadcast_to(eq[1][:, None], (eq.shape[1], 8)).astype(jnp.int32)
    q1 = params["query_gnn1"][0]
    hq = _query_gin1(xq, eq0b, eq1b, q1["lin1"]["W"], q1["lin1"]["b"],
                     q1["lin2"]["W"], q1["lin2"]["b"])

    itg = jnp.concatenate([hq, hg], axis=0)
    pad = (-n_it) % (8 * NW)
    idx_it = jnp.concatenate(
        [itedge[0], (jnp.arange(pad, dtype=jnp.int32) % nit_nodes)])
    grows = _make_row_gather(nit_nodes, n_it + pad)(itg, idx_it)[:n_it]
    dst8 = jnp.broadcast_to(itedge[1][:, None], (n_it, 8)).astype(jnp.int32)
    ip = params["interactor"][0]
    itg2, readout_g = _interactor(
        itg, grows, dst8, ip["lin1"]["W"], ip["lin1"]["b"],
        ip["lin2"]["W"], ip["lin2"]["b"],
        params["linear2"]["W"], params["linear2"]["b"],
        nq=nq, ng=ng, block=520)

    npair = npairs.T
    idx_x = jnp.concatenate([itedge[0], npair[0], itedge[1], npair[1]])
    xs = _make_row_gather(nit_nodes, 2 * (n_it + n_np))(itg2, idx_x)
    x1s = xs[:n_it + n_np]
    x2s = xs[n_it + n_np:]
    ys = jnp.concatenate([jnp.ones((n_it,), jnp.float32),
                          -jnp.ones((n_np,), jnp.float32)])

    mqb = jnp.broadcast_to(match_q[None, :],
                           (nq, match_q.shape[0])).astype(jnp.int32)
    q2 = params["query_gnn2"][0]
    hq3 = _query_gin2(itg2[:nq], mqb, eq0b, eq1b,
                      q2["lin1"]["W"], q2["lin1"]["b"],
                      q2["lin2"]["W"], q2["lin2"]["b"])

    nsub = subqueries.shape[0]
    sk0b = jnp.broadcast_to(skeleton_edges[0][:, None],
                            (skeleton_edges.shape[1], 8)).astype(jnp.int32)
    sk1b = jnp.broadcast_to(skeleton_edges[1][:, None],
                            (skeleton_edges.shape[1], 8)).astype(jnp.int32)
    ap = params["aggregate"][0]
    tp = {
        "l1w": params["linear1"]["W"], "l1b": params["linear1"]["b"].reshape(1, D),
        "l3w": params["linear3"]["W"], "l3b": params["linear3"]["b"].reshape(1, D),
        "a1wa": ap["lin1"]["W"][:D], "a1wb": ap["lin1"]["W"][D:],
        "a1b": ap["lin1"]["b"].reshape(1, D),
        "a2gw": ap["lin2"]["W"], "a2gb": ap["lin2"]["b"].reshape(1, D),
        "wwa": params["weighter"]["W"][:D], "wwb": params["weighter"]["W"][D:],
        "wwbias": params["weighter"]["b"].reshape(1, 1),
        "g2wa": params["aggregate2"]["W"][:D], "g2wb": params["aggregate2"]["W"][D:],
        "g2b": params["aggregate2"]["b"].reshape(1, D),
        "w2wa": params["weighter2"]["W"][:D], "w2wl": params["weighter2"]["W"][D:],
        "w2b": params["weighter2"]["b"].reshape(1, 1),
    }
    pj, o1 = _tail(hq3, readout_g, subqueries.astype(jnp.int32),
                   overlap_nodes.astype(jnp.int32), sk0b, sk1b, tp)
    return (o1.reshape(1), (x1s, x2s, ys), pj.reshape(nsub))


# speedup vs baseline: 1.0899x; 1.0899x over previous
import functools

import jax
import jax.numpy as jnp
from jax import lax
from jax.experimental import pallas as pl
from jax.experimental.pallas import tpu as pltpu
from jax.experimental.pallas import tpu_sc as plsc

NC, NS = 2, 16
NW = NC * NS
D = 128
_MAX_CARD, _MIN_CARD = 20.0, 0.0
_EXACT = lax.Precision.HIGHEST
_LIN = lax.Precision.DEFAULT


def _dot(a, b, prec):
    return jnp.dot(a, b, preferred_element_type=jnp.float32, precision=prec)


def _dott(a, b, prec):
    return lax.dot_general(a, b, (((0,), (0,)), ((), ())),
                           preferred_element_type=jnp.float32, precision=prec)


@functools.lru_cache(maxsize=None)
def _make_edge_agg(n_nodes, n_edges, chunk):
    assert n_edges % (NW * chunk) == 0 and n_nodes % NS == 0
    nchunk = n_edges // (NW * chunk)
    assert nchunk % 2 == 0
    rpt = n_nodes // NS
    assert rpt % chunk == 0
    nslab = rpt // chunk
    mesh = plsc.VectorSubcoreMesh(core_axis_name="c", subcore_axis_name="s")

    @functools.partial(
        pl.kernel,
        out_type=jax.ShapeDtypeStruct((NC, n_nodes, D), jnp.float32),
        mesh=mesh,
        scratch_types=[
            pltpu.VMEM((nchunk, chunk), jnp.int32),
            pltpu.VMEM((nchunk, chunk), jnp.int32),
            pltpu.VMEM((chunk, D), jnp.float32),
            pltpu.VMEM((chunk, D), jnp.float32),
            pltpu.VMEM_SHARED((n_nodes, D), jnp.float32),
            pltpu.SemaphoreType.DMA,
            pltpu.SemaphoreType.DMA,
            pltpu.SemaphoreType.DMA,
            pltpu.SemaphoreType.DMA,
        ],
    )
    def edge_agg(h_hbm, src_hbm, dst_hbm, zeros_hbm, out_hbm,
                 src_v, dst_v, rows0, rows1, agg_sh, sem0, sem1, sem2, sem3):
        cid = lax.axis_index("c")
        sid = lax.axis_index("s")
        wid = sid * NC + cid

        def slab(t):
            return pl.ds(sid * rpt + t * chunk, chunk)

        pltpu.sync_copy(zeros_hbm, rows0)
        for t in range(nslab):
            pltpu.async_copy(rows0, agg_sh.at[slab(t)], sem1)
        for t in range(nslab):
            pltpu.make_async_copy(rows0, agg_sh.at[slab(t)], sem1).wait()
        pltpu.sync_copy(src_hbm.at[wid], src_v)
        pltpu.sync_copy(dst_hbm.at[wid], dst_v)
        plsc.subcore_barrier()

        pltpu.async_copy(h_hbm.at[src_v.at[0]], rows0, sem0)
        nh = nchunk // 2

        def body(k, _):
            j0 = 2 * k
            pltpu.make_async_copy(h_hbm.at[src_v.at[j0]], rows0, sem0).wait()
            pltpu.async_copy(h_hbm.at[src_v.at[j0 + 1]], rows1, sem1)
            pltpu.async_copy(rows0, agg_sh.at[dst_v.at[j0]], sem2, add=True)
            pltpu.make_async_copy(
                h_hbm.at[src_v.at[j0 + 1]], rows1, sem1).wait()
            pltpu.async_copy(rows1, agg_sh.at[dst_v.at[j0 + 1]], sem3,
                             add=True)
            pltpu.make_async_copy(
                rows0, agg_sh.at[dst_v.at[j0]], sem2).wait()

            @pl.when(k + 1 < nh)
            def _issue_next():
                pltpu.async_copy(h_hbm.at[src_v.at[j0 + 2]], rows0, sem0)

            pltpu.make_async_copy(
                rows1, agg_sh.at[dst_v.at[j0 + 1]], sem3).wait()
            return _

        lax.fori_loop(0, nh, body, None)
        plsc.subcore_barrier()

        for t in range(nslab):
            b, sb = (rows0, sem0) if t % 2 == 0 else (rows1, sem1)
            if t >= 2:
                pltpu.make_async_copy(
                    b, out_hbm.at[cid, slab(t - 2)], sb).wait()
            pltpu.sync_copy(agg_sh.at[slab(t)], b)
            pltpu.async_copy(b, out_hbm.at[cid, slab(t)], sb)
        for t in range(max(nslab - 2, 0), nslab):
            b, sb = (rows0, sem0) if t % 2 == 0 else (rows1, sem1)
            pltpu.make_async_copy(b, out_hbm.at[cid, slab(t)], sb).wait()

    return edge_agg


@functools.lru_cache(maxsize=None)
def _make_row_gather(n_tab, n_idx):
    assert n_idx % (8 * NW) == 0
    per_w = n_idx // NW
    mesh = plsc.VectorSubcoreMesh(core_axis_name="c", subcore_axis_name="s")

    @functools.partial(
        pl.kernel,
        out_type=jax.ShapeDtypeStruct((n_idx, D), jnp.float32),
        mesh=mesh,
        scratch_types=[
            pltpu.VMEM((per_w,), jnp.int32),
            pltpu.VMEM((per_w, D), jnp.float32),
            pltpu.SemaphoreType.DMA,
        ],
    )
    def row_gather(tab_hbm, idx_hbm, out_hbm, idx_v, rows_v, sem):
        wid = lax.axis_index("s") * NC + lax.axis_index("c")
        base = wid * per_w
        pltpu.sync_copy(idx_hbm.at[pl.ds(base, per_w)], idx_v)
        pltpu.async_copy(tab_hbm.at[idx_v], rows_v, sem).wait()
        pltpu.sync_copy(rows_v, out_hbm.at[pl.ds(base, per_w)])

    return row_gather


def _mlp_sum(h, p, w1, b1, w2, b2, block):
    n = h.shape[0]

    def body(h_ref, p_ref, w1_ref, b1_ref, w2_ref, b2_ref, o_ref):
        u = h_ref[...] + p_ref[0] + p_ref[1]
        t = jnp.maximum(_dot(u, w1_ref[...], _LIN) + b1_ref[...], 0.0)
        o_ref[...] = _dot(t, w2_ref[...], _LIN) + b2_ref[...]

    return pl.pallas_call(
        body,
        grid=(n // block,),
        in_specs=[
            pl.BlockSpec((block, D), lambda i: (i, 0)),
            pl.BlockSpec((2, block, D), lambda i: (0, i, 0)),
            pl.BlockSpec((D, D), lambda i: (0, 0)),
            pl.BlockSpec((1, D), lambda i: (0, 0)),
            pl.BlockSpec((D, D), lambda i: (0, 0)),
            pl.BlockSpec((1, D), lambda i: (0, 0)),
        ],
        out_specs=pl.BlockSpec((block, D), lambda i: (i, 0)),
        out_shape=jax.ShapeDtypeStruct((n, D), jnp.float32),
    )(h, p, w1, b1.reshape(1, D), w2, b2.reshape(1, D))


def _interactor(itg, g1, g2, g3, dst8, w1, b1, w2, b2, l2w, l2b,
                nq, ng, block):
    n = itg.shape[0]
    ne = g1.shape[0]

    def body(it_ref, g1_ref, g2_ref, g3_ref, d_ref, w1_ref, b1_ref,
             w2_ref, b2_ref, l2w_ref, l2b_ref, o_ref, m_ref):
        i = pl.program_id(0)
        dst = d_ref[...][:, 0:1]
        rows = i * block + lax.broadcasted_iota(jnp.int32, (ne, block), 1)
        oh = (dst == rows).astype(jnp.float32)
        agg = (_dott(oh, g1_ref[...], _LIN) + _dott(oh, g2_ref[...], _LIN)
               ) + _dott(oh, g3_ref[...], _LIN)
        u = it_ref[...] + agg
        t = jnp.maximum(_dot(u, w1_ref[...], _LIN) + b1_ref[...], 0.0)
        y = _dot(t, w2_ref[...], _LIN) + b2_ref[...]
        o_ref[...] = y
        z = _dot(y, l2w_ref[...], _LIN) + l2b_ref[...]
        rid = i * block + lax.broadcasted_iota(jnp.int32, (block, 1), 0)
        msk = (rid >= nq).astype(jnp.float32)
        part = jnp.sum(z * msk, axis=0, keepdims=True) * (1.0 / ng)

        @pl.when(i == 0)
        def _():
            m_ref[...] = part

        @pl.when(i > 0)
        def _():
            m_ref[...] += part

    return pl.pallas_call(
        body,
        grid=(n // block,),
        in_specs=[
            pl.BlockSpec((block, D), lambda i: (i, 0)),
            pl.BlockSpec((ne, D), lambda i: (0, 0)),
            pl.BlockSpec((ne, D), lambda i: (0, 0)),
            pl.BlockSpec((ne, D), lambda i: (0, 0)),
            pl.BlockSpec((ne, 8), lambda i: (0, 0)),
            pl.BlockSpec((D, D), lambda i: (0, 0)),
            pl.BlockSpec((1, D), lambda i: (0, 0)),
            pl.BlockSpec((D, D), lambda i: (0, 0)),
            pl.BlockSpec((1, D), lambda i: (0, 0)),
            pl.BlockSpec((D, D), lambda i: (0, 0)),
            pl.BlockSpec((1, D), lambda i: (0, 0)),
        ],
        out_specs=[
            pl.BlockSpec((block, D), lambda i: (i, 0)),
            pl.BlockSpec((1, D), lambda i: (0, 0)),
        ],
        out_shape=[
            jax.ShapeDtypeStruct((n, D), jnp.float32),
            jax.ShapeDtypeStruct((1, D), jnp.float32),
        ],
    )(itg, g1, g2, g3, dst8, w1, b1.reshape(1, D), w2, b2.reshape(1, D),
      l2w, l2b.reshape(1, D))


def _query_gin1(xq, eq0b, eq1b, w1, b1, w2, b2):
    ne = eq0b.shape[0]
    nq = xq.shape[0]

    def body(x_ref, e0_ref, e1_ref, w1_ref, b1_ref, w2_ref, b2_ref, o_ref):
        src = e0_ref[...][:, 0:1]
        dst = e1_ref[...][:, 0:1]
        cols = lax.broadcasted_iota(jnp.int32, (ne, nq), 1)
        oh_s = (src == cols).astype(jnp.float32)
        oh_d = (dst == cols).astype(jnp.float32)
        x = x_ref[...]
        g = _dot(oh_s, x, _EXACT)
        agg = _dott(oh_d, g, _EXACT)
        u = x + agg
        t = jnp.maximum(_dot(u, w1_ref[...], _LIN) + b1_ref[...], 0.0)
        o_ref[...] = _dot(t, w2_ref[...], _LIN) + b2_ref[...]

    return pl.pallas_call(
        body,
        out_shape=jax.ShapeDtypeStruct((nq, D), jnp.float32),
    )(xq, eq0b, eq1b, w1, b1.reshape(1, D), w2, b2.reshape(1, D))


def _query_gin2(hq, mqb, eq0b, eq1b, w1, b1, w2, b2):
    ne = eq0b.shape[0]
    nq = hq.shape[0]
    nm = mqb.shape[1]

    def body(h_ref, mq_ref, e0_ref, e1_ref, w1_ref, b1_ref, w2_ref, b2_ref,
             o_ref):
        rid = lax.broadcasted_iota(jnp.int32, (nq, nm), 0)
        hits = (mq_ref[...] == rid).astype(jnp.float32)
        cnt = jnp.sum(hits, axis=1, keepdims=True)
        msk = (cnt == 0.0).astype(jnp.float32)
        hm = h_ref[...] * msk
        src = e0_ref[...][:, 0:1]
        dst = e1_ref[...][:, 0:1]
        cols = lax.broadcasted_iota(jnp.int32, (ne, nq), 1)
        oh_s = (src == cols).astype(jnp.float32)
        oh_d = (dst == cols).astype(jnp.float32)
        g = _dot(oh_s, hm, _EXACT)
        agg = _dott(oh_d, g, _EXACT)
        u = hm + agg
        t = jnp.maximum(_dot(u, w1_ref[...], _LIN) + b1_ref[...], 0.0)
        y = _dot(t, w2_ref[...], _LIN) + b2_ref[...]
        z = hm + y
        o_ref[...] = z * jax.nn.sigmoid(z)

    return pl.pallas_call(
        body,
        out_shape=jax.ShapeDtypeStruct((nq, D), jnp.float32),
    )(hq, mqb, eq0b, eq1b, w1, b1.reshape(1, D), w2, b2.reshape(1, D))


def _tail(hq3, readout_g, subq, ovn, sk0b, sk1b, tp):
    nsub, sublen = subq.shape
    nov, ovlen = ovn.shape
    nsk_e = sk0b.shape[0]

    def body(hq_ref, rg_ref, sq_ref, ov_ref, s0_ref, s1_ref,
             l1w, l1b, l3w, l3b,
             a1wa, a1wb, a1b, a2gw, a2gb,
             wwa, wwb, wwbias, g2wa, g2wb, g2b, w2wa, w2wl, w2b,
             pj_ref, o1_ref):
        hq = hq_ref[...]
        readout_g = rg_ref[...]
        z1 = _dot(hq, l1w[...], _LIN) + l1b[...]
        aq = jnp.zeros((nsub, D), jnp.float32)
        colsq = lax.broadcasted_iota(jnp.int32, (nsub, D), 1)
        for j in range(sublen):
            aq = aq + (sq_ref[...][:, j:j + 1] == colsq).astype(jnp.float32)
        readout_q = _dot(aq * (1.0 / sublen), z1, _EXACT)
        z3 = _dot(hq, l3w[...], _LIN) + l3b[...]
        ao = jnp.zeros((nov, D), jnp.float32)
        colso = lax.broadcasted_iota(jnp.int32, (nov, D), 1)
        for j in range(ovlen):
            ao = ao + (ov_ref[...][:, j:j + 1] == colso).astype(jnp.float32)
        ov_feat = _dot(ao * (1.0 / ovlen), z3, _EXACT)
        colss = lax.broadcasted_iota(jnp.int32, (nsk_e, nsub), 1)
        g0 = (s0_ref[...][:, 0:1] == colss).astype(jnp.float32)
        g1 = (s1_ref[...][:, 0:1] == colss).astype(jnp.float32)
        gsum = g0 + g1
        ovf = _dott(gsum, ov_feat, _EXACT)
        cnts = 1.0 + _dott(gsum, jnp.ones((nsk_e, 1), jnp.float32), _EXACT)
        ovf = ovf / cnts
        xg_l = _dot(g0, readout_q, _EXACT)
        xg_r = _dot(g0, ovf, _EXACT)
        agg_l = _dott(g1, xg_l, _EXACT)
        agg_r = _dott(g1, xg_r, _EXACT)
        u_l = readout_q + agg_l
        u_r = ovf + agg_r
        t = (_dot(u_l, a1wa[...], _LIN) + _dot(u_r, a1wb[...], _LIN)
             + a1b[...])
        t = jnp.maximum(t, 0.0)
        ovl2 = _dot(t, a2gw[...], _LIN) + a2gb[...]
        wl = (_dot(readout_q, wwa[...], _LIN) + _dot(ovl2, wwb[...], _LIN)
              + wwbias[...])
        wmax = jnp.max(wl, axis=0, keepdims=True)
        we = jnp.exp(wl - wmax)
        wsm = we / jnp.sum(we, axis=0, keepdims=True)
        hsk = readout_q * wsm
        rq2 = jnp.mean(hsk, axis=0, keepdims=True)
        nsk = jnp.sqrt(jnp.sum(hsk * hsk, axis=1, keepdims=True))
        s = jnp.sum(hsk * readout_g, axis=1, keepdims=True)
        anyz = jnp.max((nsk == 0.0).astype(jnp.float32), axis=0,
                       keepdims=True)
        pj = jnp.where(anyz > 0.5, jnp.zeros_like(s),
                       s / jnp.where(nsk == 0.0, 1.0, nsk))
        pj_ref[...] = pj
        swl = rq2 * jax.nn.sigmoid(rq2)
        swr = readout_g * jax.nn.sigmoid(readout_g)
        ro = (_dot(swl, g2wa[...], _LIN) + _dot(swr, g2wb[...], _LIN)
              + g2b[...])
        o = _dot(ro, w2wa[...], _LIN) + 8192.0 * w2wl[...] + w2b[...]
        o1_ref[...] = _MIN_CARD + (_MAX_CARD - _MIN_CARD) * jax.nn.sigmoid(o)

    return pl.pallas_call(
        body,
        out_shape=[
            jax.ShapeDtypeStruct((nsub, 1), jnp.float32),
            jax.ShapeDtypeStruct((1, 1), jnp.float32),
        ],
    )(hq3, readout_g, subq, ovn, sk0b, sk1b,
      tp["l1w"], tp["l1b"], tp["l3w"], tp["l3b"],
      tp["a1wa"], tp["a1wb"], tp["a1b"], tp["a2gw"], tp["a2gb"],
      tp["wwa"], tp["wwb"], tp["wwbias"],
      tp["g2wa"], tp["g2wb"], tp["g2b"], tp["w2wa"], tp["w2wl"], tp["w2b"])


def kernel(xg, eg, xq, eq, itedge, npairs, match_q, subqueries,
           skeleton_edges, overlap_nodes, params):
    ng, nq = xg.shape[0], xq.shape[0]
    n_eg = eg.shape[1]
    n_it = itedge.shape[1]
    n_np = npairs.shape[0]
    nit_nodes = nq + ng

    chunk = 128
    nchunk = n_eg // (NW * chunk)
    src_r = eg[0].reshape(NW, nchunk, chunk)
    dst_r = eg[1].reshape(NW, nchunk, chunk)
    zeros_g = jnp.zeros((chunk, D), jnp.float32)

    gp = params["graph_gnn"]
    edge_agg = _make_edge_agg(ng, n_eg, chunk)

    p1 = edge_agg(xg, src_r, dst_r, zeros_g)
    h1 = _mlp_sum(xg, p1, gp[0]["lin1"]["W"], gp[0]["lin1"]["b"],
                  gp[0]["lin2"]["W"], gp[0]["lin2"]["b"], block=2048)
    p2 = edge_agg(h1, src_r, dst_r, zeros_g)
    hg = _mlp_sum(h1, p2, gp[1]["lin1"]["W"], gp[1]["lin1"]["b"],
                  gp[1]["lin2"]["W"], gp[1]["lin2"]["b"], block=2048)

    eq0b = jnp.broadcast_to(eq[0][:, None], (eq.shape[1], 8)).astype(jnp.int32)
    eq1b = jnp.broadcast_to(eq[1][:, None], (eq.shape[1], 8)).astype(jnp.int32)
    q1 = params["query_gnn1"][0]
    hq = _query_gin1(xq, eq0b, eq1b, q1["lin1"]["W"], q1["lin1"]["b"],
                     q1["lin2"]["W"], q1["lin2"]["b"])

    itg = jnp.concatenate([hq, hg], axis=0)
    pad = (-n_it) % (8 * NW)
    idx_it = jnp.concatenate(
        [itedge[0], (jnp.arange(pad, dtype=jnp.int32) % nit_nodes)])
    grows = _make_row_gather(nit_nodes, n_it + pad)(itg, idx_it)[:n_it]
    g1 = grows.astype(jnp.bfloat16).astype(jnp.float32)
    r = grows - g1
    g2 = r.astype(jnp.bfloat16).astype(jnp.float32)
    g3 = r - g2
    dst8 = jnp.broadcast_to(itedge[1][:, None], (n_it, 8)).astype(jnp.int32)
    ip = params["interactor"][0]
    itg2, readout_g = _interactor(
        itg, g1, g2, g3, dst8, ip["lin1"]["W"], ip["lin1"]["b"],
        ip["lin2"]["W"], ip["lin2"]["b"],
        params["linear2"]["W"], params["linear2"]["b"],
        nq=nq, ng=ng, block=520)

    npair = npairs.T
    idx_x = jnp.concatenate([itedge[0], npair[0], itedge[1], npair[1]])
    xs = _make_row_gather(nit_nodes, 2 * (n_it + n_np))(itg2, idx_x)
    x1s = xs[:n_it + n_np]
    x2s = xs[n_it + n_np:]
    ys = jnp.concatenate([jnp.ones((n_it,), jnp.float32),
                          -jnp.ones((n_np,), jnp.float32)])

    mqb = jnp.broadcast_to(match_q[None, :],
                           (nq, match_q.shape[0])).astype(jnp.int32)
    q2 = params["query_gnn2"][0]
    hq3 = _query_gin2(itg2[:nq], mqb, eq0b, eq1b,
                      q2["lin1"]["W"], q2["lin1"]["b"],
                      q2["lin2"]["W"], q2["lin2"]["b"])

    nsub = subqueries.shape[0]
    sk0b = jnp.broadcast_to(skeleton_edges[0][:, None],
                            (skeleton_edges.shape[1], 8)).astype(jnp.int32)
    sk1b = jnp.broadcast_to(skeleton_edges[1][:, None],
                            (skeleton_edges.shape[1], 8)).astype(jnp.int32)
    ap = params["aggregate"][0]
    tp = {
        "l1w": params["linear1"]["W"], "l1b": params["linear1"]["b"].reshape(1, D),
        "l3w": params["linear3"]["W"], "l3b": params["linear3"]["b"].reshape(1, D),
        "a1wa": ap["lin1"]["W"][:D], "a1wb": ap["lin1"]["W"][D:],
        "a1b": ap["lin1"]["b"].reshape(1, D),
        "a2gw": ap["lin2"]["W"], "a2gb": ap["lin2"]["b"].reshape(1, D),
        "wwa": params["weighter"]["W"][:D], "wwb": params["weighter"]["W"][D:],
        "wwbias": params["weighter"]["b"].reshape(1, 1),
        "g2wa": params["aggregate2"]["W"][:D], "g2wb": params["aggregate2"]["W"][D:],
        "g2b": params["aggregate2"]["b"].reshape(1, D),
        "w2wa": params["weighter2"]["W"][:D], "w2wl": params["weighter2"]["W"][D:],
        "w2b": params["weighter2"]["b"].reshape(1, 1),
    }
    pj, o1 = _tail(hq3, readout_g, subqueries.astype(jnp.int32),
                   overlap_nodes.astype(jnp.int32), sk0b, sk1b, tp)
    return (o1.reshape(1), (x1s, x2s, ys), pj.reshape(nsub))


# speedup vs baseline: 1.1082x; 1.0168x over previous
import functools

import jax
import jax.numpy as jnp
from jax import lax
from jax.experimental import pallas as pl
from jax.experimental.pallas import tpu as pltpu
from jax.experimental.pallas import tpu_sc as plsc

NC, NS = 2, 16
NW = NC * NS
D = 128
_MAX_CARD, _MIN_CARD = 20.0, 0.0
_EXACT = lax.Precision.HIGHEST
_LIN = lax.Precision.DEFAULT


def _dot(a, b, prec):
    return jnp.dot(a, b, preferred_element_type=jnp.float32, precision=prec)


def _dott(a, b, prec):
    return lax.dot_general(a, b, (((0,), (0,)), ((), ())),
                           preferred_element_type=jnp.float32, precision=prec)


@functools.lru_cache(maxsize=None)
def _make_edge_agg(n_nodes, n_edges, chunk):
    assert n_edges % (NW * chunk) == 0 and n_nodes % NS == 0
    nchunk = n_edges // (NW * chunk)
    assert nchunk % 2 == 0
    rpt = n_nodes // NS
    assert rpt % chunk == 0
    nslab = rpt // chunk
    mesh = plsc.VectorSubcoreMesh(core_axis_name="c", subcore_axis_name="s")

    @functools.partial(
        pl.kernel,
        out_type=jax.ShapeDtypeStruct((NC, n_nodes, D), jnp.float32),
        mesh=mesh,
        scratch_types=[
            pltpu.VMEM((nchunk, chunk), jnp.int32),
            pltpu.VMEM((nchunk, chunk), jnp.int32),
            pltpu.VMEM((chunk, D), jnp.float32),
            pltpu.VMEM((chunk, D), jnp.float32),
            pltpu.VMEM_SHARED((n_nodes, D), jnp.float32),
            pltpu.SemaphoreType.DMA,
            pltpu.SemaphoreType.DMA,
            pltpu.SemaphoreType.DMA,
            pltpu.SemaphoreType.DMA,
        ],
    )
    def edge_agg(h_hbm, src_hbm, dst_hbm, zeros_hbm, out_hbm,
                 src_v, dst_v, rows0, rows1, agg_sh, sem0, sem1, sem2, sem3):
        cid = lax.axis_index("c")
        sid = lax.axis_index("s")
        wid = sid * NC + cid

        def slab(t):
            return pl.ds(sid * rpt + t * chunk, chunk)

        pltpu.sync_copy(zeros_hbm, rows0)
        for t in range(nslab):
            pltpu.async_copy(rows0, agg_sh.at[slab(t)], sem1)
        for t in range(nslab):
            pltpu.make_async_copy(rows0, agg_sh.at[slab(t)], sem1).wait()
        pltpu.sync_copy(src_hbm.at[wid], src_v)
        pltpu.sync_copy(dst_hbm.at[wid], dst_v)
        plsc.subcore_barrier()

        pltpu.async_copy(h_hbm.at[src_v.at[0]], rows0, sem0)
        nh = nchunk // 2

        def body(k, _):
            j0 = 2 * k
            pltpu.make_async_copy(h_hbm.at[src_v.at[j0]], rows0, sem0).wait()
            pltpu.async_copy(h_hbm.at[src_v.at[j0 + 1]], rows1, sem1)
            pltpu.async_copy(rows0, agg_sh.at[dst_v.at[j0]], sem2, add=True)
            pltpu.make_async_copy(
                h_hbm.at[src_v.at[j0 + 1]], rows1, sem1).wait()
            pltpu.async_copy(rows1, agg_sh.at[dst_v.at[j0 + 1]], sem3,
                             add=True)
            pltpu.make_async_copy(
                rows0, agg_sh.at[dst_v.at[j0]], sem2).wait()

            @pl.when(k + 1 < nh)
            def _issue_next():
                pltpu.async_copy(h_hbm.at[src_v.at[j0 + 2]], rows0, sem0)

            pltpu.make_async_copy(
                rows1, agg_sh.at[dst_v.at[j0 + 1]], sem3).wait()
            return _

        lax.fori_loop(0, nh, body, None)
        plsc.subcore_barrier()

        for t in range(nslab):
            b, sb = (rows0, sem0) if t % 2 == 0 else (rows1, sem1)
            if t >= 2:
                pltpu.make_async_copy(
                    b, out_hbm.at[cid, slab(t - 2)], sb).wait()
            pltpu.sync_copy(agg_sh.at[slab(t)], b)
            pltpu.async_copy(b, out_hbm.at[cid, slab(t)], sb)
        for t in range(max(nslab - 2, 0), nslab):
            b, sb = (rows0, sem0) if t % 2 == 0 else (rows1, sem1)
            pltpu.make_async_copy(b, out_hbm.at[cid, slab(t)], sb).wait()

    return edge_agg


@functools.lru_cache(maxsize=None)
def _make_row_gather(n_tab, n_idx):
    assert n_idx % (8 * NW) == 0
    per_w = n_idx // NW
    mesh = plsc.VectorSubcoreMesh(core_axis_name="c", subcore_axis_name="s")

    @functools.partial(
        pl.kernel,
        out_type=jax.ShapeDtypeStruct((n_idx, D), jnp.float32),
        mesh=mesh,
        scratch_types=[
            pltpu.VMEM((per_w,), jnp.int32),
            pltpu.VMEM((per_w, D), jnp.float32),
            pltpu.SemaphoreType.DMA,
        ],
    )
    def row_gather(tab_hbm, idx_hbm, out_hbm, idx_v, rows_v, sem):
        wid = lax.axis_index("s") * NC + lax.axis_index("c")
        base = wid * per_w
        pltpu.sync_copy(idx_hbm.at[pl.ds(base, per_w)], idx_v)
        pltpu.async_copy(tab_hbm.at[idx_v], rows_v, sem).wait()
        pltpu.sync_copy(rows_v, out_hbm.at[pl.ds(base, per_w)])

    return row_gather


@functools.lru_cache(maxsize=None)
def _make_row_gather2(n_tab, n_half):
    assert n_half % (8 * NS) == 0
    per_w = n_half // NS
    mesh = plsc.VectorSubcoreMesh(core_axis_name="c", subcore_axis_name="s")

    @functools.partial(
        pl.kernel,
        out_type=[jax.ShapeDtypeStruct((n_half, D), jnp.float32),
                  jax.ShapeDtypeStruct((n_half, D), jnp.float32)],
        mesh=mesh,
        scratch_types=[
            pltpu.VMEM((per_w,), jnp.int32),
            pltpu.VMEM((per_w, D), jnp.float32),
            pltpu.SemaphoreType.DMA,
        ],
    )
    def row_gather2(tab_hbm, idx_hbm, o1_hbm, o2_hbm, idx_v, rows_v, sem):
        wid = lax.axis_index("s") * NC + lax.axis_index("c")
        base = wid * per_w
        pltpu.sync_copy(idx_hbm.at[pl.ds(base, per_w)], idx_v)
        pltpu.async_copy(tab_hbm.at[idx_v], rows_v, sem).wait()

        @pl.when(wid < NS)
        def _():
            pltpu.sync_copy(rows_v, o1_hbm.at[pl.ds(base, per_w)])

        @pl.when(wid >= NS)
        def _():
            pltpu.sync_copy(
                rows_v, o2_hbm.at[pl.ds(base - n_half, per_w)])

    return row_gather2


def _mlp_sum(h, p, w1, b1, w2, b2, block):
    n = h.shape[0]

    def body(h_ref, p_ref, w1_ref, b1_ref, w2_ref, b2_ref, o_ref):
        u = h_ref[...] + p_ref[0] + p_ref[1]
        t = jnp.maximum(_dot(u, w1_ref[...], _LIN) + b1_ref[...], 0.0)
        o_ref[...] = _dot(t, w2_ref[...], _LIN) + b2_ref[...]

    return pl.pallas_call(
        body,
        grid=(n // block,),
        in_specs=[
            pl.BlockSpec((block, D), lambda i: (i, 0)),
            pl.BlockSpec((2, block, D), lambda i: (0, i, 0)),
            pl.BlockSpec((D, D), lambda i: (0, 0)),
            pl.BlockSpec((1, D), lambda i: (0, 0)),
            pl.BlockSpec((D, D), lambda i: (0, 0)),
            pl.BlockSpec((1, D), lambda i: (0, 0)),
        ],
        out_specs=pl.BlockSpec((block, D), lambda i: (i, 0)),
        out_shape=jax.ShapeDtypeStruct((n, D), jnp.float32),
    )(h, p, w1, b1.reshape(1, D), w2, b2.reshape(1, D))


def _interactor(itg, g1, g2, g3, dst8, w1, b1, w2, b2, l2w, l2b,
                nq, ng, block):
    n = itg.shape[0]
    ne = g1.shape[0]

    def body(it_ref, g1_ref, g2_ref, g3_ref, d_ref, w1_ref, b1_ref,
             w2_ref, b2_ref, l2w_ref, l2b_ref, o_ref, m_ref):
        i = pl.program_id(0)
        dst = d_ref[...][:, 0:1]
        rows = i * block + lax.broadcasted_iota(jnp.int32, (ne, block), 1)
        oh = (dst == rows).astype(jnp.float32)
        agg = (_dott(oh, g1_ref[...], _LIN) + _dott(oh, g2_ref[...], _LIN)
               ) + _dott(oh, g3_ref[...], _LIN)
        u = it_ref[...] + agg
        t = jnp.maximum(_dot(u, w1_ref[...], _LIN) + b1_ref[...], 0.0)
        y = _dot(t, w2_ref[...], _LIN) + b2_ref[...]
        o_ref[...] = y
        z = _dot(y, l2w_ref[...], _LIN) + l2b_ref[...]
        rid = i * block + lax.broadcasted_iota(jnp.int32, (block, 1), 0)
        msk = (rid >= nq).astype(jnp.float32)
        part = jnp.sum(z * msk, axis=0, keepdims=True) * (1.0 / ng)

        @pl.when(i == 0)
        def _():
            m_ref[...] = part

        @pl.when(i > 0)
        def _():
            m_ref[...] += part

    return pl.pallas_call(
        body,
        grid=(n // block,),
        in_specs=[
            pl.BlockSpec((block, D), lambda i: (i, 0)),
            pl.BlockSpec((ne, D), lambda i: (0, 0)),
            pl.BlockSpec((ne, D), lambda i: (0, 0)),
            pl.BlockSpec((ne, D), lambda i: (0, 0)),
            pl.BlockSpec((ne, 8), lambda i: (0, 0)),
            pl.BlockSpec((D, D), lambda i: (0, 0)),
            pl.BlockSpec((1, D), lambda i: (0, 0)),
            pl.BlockSpec((D, D), lambda i: (0, 0)),
            pl.BlockSpec((1, D), lambda i: (0, 0)),
            pl.BlockSpec((D, D), lambda i: (0, 0)),
            pl.BlockSpec((1, D), lambda i: (0, 0)),
        ],
        out_specs=[
            pl.BlockSpec((block, D), lambda i: (i, 0)),
            pl.BlockSpec((1, D), lambda i: (0, 0)),
        ],
        out_shape=[
            jax.ShapeDtypeStruct((n, D), jnp.float32),
            jax.ShapeDtypeStruct((1, D), jnp.float32),
        ],
    )(itg, g1, g2, g3, dst8, w1, b1.reshape(1, D), w2, b2.reshape(1, D),
      l2w, l2b.reshape(1, D))


def _query_gin1(xq, eq0b, eq1b, w1, b1, w2, b2):
    ne = eq0b.shape[0]
    nq = xq.shape[0]

    def body(x_ref, e0_ref, e1_ref, w1_ref, b1_ref, w2_ref, b2_ref, o_ref):
        src = e0_ref[...][:, 0:1]
        dst = e1_ref[...][:, 0:1]
        cols = lax.broadcasted_iota(jnp.int32, (ne, nq), 1)
        oh_s = (src == cols).astype(jnp.float32)
        oh_d = (dst == cols).astype(jnp.float32)
        x = x_ref[...]
        g = _dot(oh_s, x, _EXACT)
        agg = _dott(oh_d, g, _EXACT)
        u = x + agg
        t = jnp.maximum(_dot(u, w1_ref[...], _LIN) + b1_ref[...], 0.0)
        o_ref[...] = _dot(t, w2_ref[...], _LIN) + b2_ref[...]

    return pl.pallas_call(
        body,
        out_shape=jax.ShapeDtypeStruct((nq, D), jnp.float32),
    )(xq, eq0b, eq1b, w1, b1.reshape(1, D), w2, b2.reshape(1, D))


def _query_gin2(hq, mqb, eq0b, eq1b, w1, b1, w2, b2):
    ne = eq0b.shape[0]
    nq, nm = mqb.shape

    def body(h_ref, mq_ref, e0_ref, e1_ref, w1_ref, b1_ref, w2_ref, b2_ref,
             o_ref):
        rid = lax.broadcasted_iota(jnp.int32, (nq, nm), 0)
        hits = (mq_ref[...] == rid).astype(jnp.float32)
        cnt = jnp.sum(hits, axis=1, keepdims=True)
        msk = (cnt == 0.0).astype(jnp.float32)
        hm = h_ref[...] * msk
        src = e0_ref[...][:, 0:1]
        dst = e1_ref[...][:, 0:1]
        cols = lax.broadcasted_iota(jnp.int32, (ne, nq), 1)
        oh_s = (src == cols).astype(jnp.float32)
        oh_d = (dst == cols).astype(jnp.float32)
        g = _dot(oh_s, hm, _EXACT)
        agg = _dott(oh_d, g, _EXACT)
        u = hm + agg
        t = jnp.maximum(_dot(u, w1_ref[...], _LIN) + b1_ref[...], 0.0)
        y = _dot(t, w2_ref[...], _LIN) + b2_ref[...]
        z = hm + y
        o_ref[...] = z * jax.nn.sigmoid(z)

    return pl.pallas_call(
        body,
        grid=(1,),
        in_specs=[
            pl.BlockSpec((nq, D), lambda i: (0, 0)),
            pl.BlockSpec(mqb.shape, lambda i: (0, 0)),
            pl.BlockSpec((ne, 8), lambda i: (0, 0)),
            pl.BlockSpec((ne, 8), lambda i: (0, 0)),
            pl.BlockSpec((D, D), lambda i: (0, 0)),
            pl.BlockSpec((1, D), lambda i: (0, 0)),
            pl.BlockSpec((D, D), lambda i: (0, 0)),
            pl.BlockSpec((1, D), lambda i: (0, 0)),
        ],
        out_specs=pl.BlockSpec((nq, D), lambda i: (0, 0)),
        out_shape=jax.ShapeDtypeStruct((nq, D), jnp.float32),
    )(hq, mqb, eq0b, eq1b, w1, b1.reshape(1, D), w2, b2.reshape(1, D))


def _tail(hq3, readout_g, subq, ovn, sk0b, sk1b, tp):
    nsub, sublen = subq.shape
    nov, ovlen = ovn.shape
    nsk_e = sk0b.shape[0]

    def body(hq_ref, rg_ref, sq_ref, ov_ref, s0_ref, s1_ref,
             l1w, l1b, l3w, l3b,
             a1wa, a1wb, a1b, a2gw, a2gb,
             wwa, wwb, wwbias, g2wa, g2wb, g2b, w2wa, w2wl, w2b,
             pj_ref, o1_ref):
        hq = hq_ref[...]
        readout_g = rg_ref[...]
        z1 = _dot(hq, l1w[...], _LIN) + l1b[...]
        aq = jnp.zeros((nsub, D), jnp.float32)
        colsq = lax.broadcasted_iota(jnp.int32, (nsub, D), 1)
        for j in range(sublen):
            aq = aq + (sq_ref[...][:, j:j + 1] == colsq).astype(jnp.float32)
        readout_q = _dot(aq * (1.0 / sublen), z1, _EXACT)
        z3 = _dot(hq, l3w[...], _LIN) + l3b[...]
        ao = jnp.zeros((nov, D), jnp.float32)
        colso = lax.broadcasted_iota(jnp.int32, (nov, D), 1)
        for j in range(ovlen):
            ao = ao + (ov_ref[...][:, j:j + 1] == colso).astype(jnp.float32)
        ov_feat = _dot(ao * (1.0 / ovlen), z3, _EXACT)
        colss = lax.broadcasted_iota(jnp.int32, (nsk_e, nsub), 1)
        g0 = (s0_ref[...][:, 0:1] == colss).astype(jnp.float32)
        g1 = (s1_ref[...][:, 0:1] == colss).astype(jnp.float32)
        gsum = g0 + g1
        ovf = _dott(gsum, ov_feat, _EXACT)
        cnts = 1.0 + _dott(gsum, jnp.ones((nsk_e, 1), jnp.float32), _EXACT)
        ovf = ovf / cnts
        xg_l = _dot(g0, readout_q, _EXACT)
        xg_r = _dot(g0, ovf, _EXACT)
        agg_l = _dott(g1, xg_l, _EXACT)
        agg_r = _dott(g1, xg_r, _EXACT)
        u_l = readout_q + agg_l
        u_r = ovf + agg_r
        t = (_dot(u_l, a1wa[...], _LIN) + _dot(u_r, a1wb[...], _LIN)
             + a1b[...])
        t = jnp.maximum(t, 0.0)
        ovl2 = _dot(t, a2gw[...], _LIN) + a2gb[...]
        wl = (_dot(readout_q, wwa[...], _LIN) + _dot(ovl2, wwb[...], _LIN)
              + wwbias[...])
        wmax = jnp.max(wl, axis=0, keepdims=True)
        we = jnp.exp(wl - wmax)
        wsm = we / jnp.sum(we, axis=0, keepdims=True)
        hsk = readout_q * wsm
        rq2 = jnp.mean(hsk, axis=0, keepdims=True)
        nsk = jnp.sqrt(jnp.sum(hsk * hsk, axis=1, keepdims=True))
        s = jnp.sum(hsk * readout_g, axis=1, keepdims=True)
        anyz = jnp.max((nsk == 0.0).astype(jnp.float32), axis=0,
                       keepdims=True)
        pj = jnp.where(anyz > 0.5, jnp.zeros_like(s),
                       s / jnp.where(nsk == 0.0, 1.0, nsk))
        pj_ref[...] = pj
        swl = rq2 * jax.nn.sigmoid(rq2)
        swr = readout_g * jax.nn.sigmoid(readout_g)
        ro = (_dot(swl, g2wa[...], _LIN) + _dot(swr, g2wb[...], _LIN)
              + g2b[...])
        o = _dot(ro, w2wa[...], _LIN) + 8192.0 * w2wl[...] + w2b[...]
        o1_ref[...] = _MIN_CARD + (_MAX_CARD - _MIN_CARD) * jax.nn.sigmoid(o)

    return pl.pallas_call(
        body,
        out_shape=[
            jax.ShapeDtypeStruct((nsub, 1), jnp.float32),
            jax.ShapeDtypeStruct((1, 1), jnp.float32),
        ],
    )(hq3, readout_g, subq, ovn, sk0b, sk1b,
      tp["l1w"], tp["l1b"], tp["l3w"], tp["l3b"],
      tp["a1wa"], tp["a1wb"], tp["a1b"], tp["a2gw"], tp["a2gb"],
      tp["wwa"], tp["wwb"], tp["wwbias"],
      tp["g2wa"], tp["g2wb"], tp["g2b"], tp["w2wa"], tp["w2wl"], tp["w2b"])


def kernel(xg, eg, xq, eq, itedge, npairs, match_q, subqueries,
           skeleton_edges, overlap_nodes, params):
    ng, nq = xg.shape[0], xq.shape[0]
    n_eg = eg.shape[1]
    n_it = itedge.shape[1]
    n_np = npairs.shape[0]
    nit_nodes = nq + ng

    chunk = 128
    nchunk = n_eg // (NW * chunk)
    src_r = eg[0].reshape(NW, nchunk, chunk)
    dst_r = eg[1].reshape(NW, nchunk, chunk)
    zeros_g = jnp.zeros((chunk, D), jnp.float32)

    gp = params["graph_gnn"]
    edge_agg = _make_edge_agg(ng, n_eg, chunk)

    p1 = edge_agg(xg, src_r, dst_r, zeros_g)
    h1 = _mlp_sum(xg, p1, gp[0]["lin1"]["W"], gp[0]["lin1"]["b"],
                  gp[0]["lin2"]["W"], gp[0]["lin2"]["b"], block=2048)
    p2 = edge_agg(h1, src_r, dst_r, zeros_g)
    hg = _mlp_sum(h1, p2, gp[1]["lin1"]["W"], gp[1]["lin1"]["b"],
                  gp[1]["lin2"]["W"], gp[1]["lin2"]["b"], block=2048)

    eq0b = jnp.broadcast_to(eq[0][:, None], (eq.shape[1], 8)).astype(jnp.int32)
    eq1b = jnp.broadcast_to(eq[1][:, None], (eq.shape[1], 8)).astype(jnp.int32)
    q1 = params["query_gnn1"][0]
    hq = _query_gin1(xq, eq0b, eq1b, q1["lin1"]["W"], q1["lin1"]["b"],
                     q1["lin2"]["W"], q1["lin2"]["b"])

    itg = jnp.concatenate([hq, hg], axis=0)
    pad = (-n_it) % (8 * NW)
    idx_it = jnp.concatenate(
        [itedge[0], (jnp.arange(pad, dtype=jnp.int32) % nit_nodes)])
    grows = _make_row_gather(nit_nodes, n_it + pad)(itg, idx_it)[:n_it]
    g1 = grows.astype(jnp.bfloat16).astype(jnp.float32)
    r = grows - g1
    g2 = r.astype(jnp.bfloat16).astype(jnp.float32)
    g3 = r - g2
    dst8 = jnp.broadcast_to(itedge[1][:, None], (n_it, 8)).astype(jnp.int32)
    ip = params["interactor"][0]
    itg2, readout_g = _interactor(
        itg, g1, g2, g3, dst8, ip["lin1"]["W"], ip["lin1"]["b"],
        ip["lin2"]["W"], ip["lin2"]["b"],
        params["linear2"]["W"], params["linear2"]["b"],
        nq=nq, ng=ng, block=520)

    npair = npairs.T
    idx_x = jnp.concatenate([itedge[0], npair[0], itedge[1], npair[1]])
    x1s, x2s = _make_row_gather2(nit_nodes, n_it + n_np)(itg2, idx_x)
    ys = jnp.concatenate([jnp.ones((n_it,), jnp.float32),
                          -jnp.ones((n_np,), jnp.float32)])

    mqb = jnp.broadcast_to(match_q[None, :],
                           (nq, match_q.shape[0])).astype(jnp.int32)
    q2 = params["query_gnn2"][0]
    hq3 = _query_gin2(itg2, mqb, eq0b, eq1b,
                      q2["lin1"]["W"], q2["lin1"]["b"],
                      q2["lin2"]["W"], q2["lin2"]["b"])

    nsub = subqueries.shape[0]
    sk0b = jnp.broadcast_to(skeleton_edges[0][:, None],
                            (skeleton_edges.shape[1], 8)).astype(jnp.int32)
    sk1b = jnp.broadcast_to(skeleton_edges[1][:, None],
                            (skeleton_edges.shape[1], 8)).astype(jnp.int32)
    ap = params["aggregate"][0]
    tp = {
        "l1w": params["linear1"]["W"], "l1b": params["linear1"]["b"].reshape(1, D),
        "l3w": params["linear3"]["W"], "l3b": params["linear3"]["b"].reshape(1, D),
        "a1wa": ap["lin1"]["W"][:D], "a1wb": ap["lin1"]["W"][D:],
        "a1b": ap["lin1"]["b"].reshape(1, D),
        "a2gw": ap["lin2"]["W"], "a2gb": ap["lin2"]["b"].reshape(1, D),
        "wwa": params["weighter"]["W"][:D], "wwb": params["weighter"]["W"][D:],
        "wwbias": params["weighter"]["b"].reshape(1, 1),
        "g2wa": params["aggregate2"]["W"][:D], "g2wb": params["aggregate2"]["W"][D:],
        "g2b": params["aggregate2"]["b"].reshape(1, D),
        "w2wa": params["weighter2"]["W"][:D], "w2wl": params["weighter2"]["W"][D:],
        "w2b": params["weighter2"]["b"].reshape(1, 1),
    }
    pj, o1 = _tail(hq3, readout_g, subqueries.astype(jnp.int32),
                   overlap_nodes.astype(jnp.int32), sk0b, sk1b, tp)
    return (o1.reshape(1), (x1s, x2s, ys), pj.reshape(nsub))
